# baseline (speedup 1.0000x reference)
import functools

import jax
import jax.numpy as jnp
from jax import lax
from jax.experimental import pallas as pl
from jax.experimental.pallas import tpu as pltpu
from jax.experimental.pallas import tpu_sc as plsc

_NUM_CLASSES = 100000
_FEAT_DIM = 64
_BATCH = 16384
_LAMBDA_C = 0.003

_NC = 2
_NS = 16
_NL = 16
_NW = _NC * _NS
_B_PER_W = _BATCH // _NW
_GROUPS = _FEAT_DIM // _NL


def _partials(features, labels, centers):
    mesh = plsc.VectorSubcoreMesh(core_axis_name="c", subcore_axis_name="s")

    @functools.partial(
        pl.kernel,
        mesh=mesh,
        out_type=jax.ShapeDtypeStruct((_NW, _NL), jnp.float32),
        compiler_params=pltpu.CompilerParams(use_tc_tiling_on_sc=False),
        scratch_types=[
            pltpu.VMEM((_B_PER_W,), jnp.int32),
            pltpu.VMEM((_B_PER_W, _FEAT_DIM), jnp.float32),
            pltpu.VMEM((_B_PER_W, _FEAT_DIM), jnp.float32),
            pltpu.VMEM((_NL,), jnp.float32),
            pltpu.SemaphoreType.DMA,
            pltpu.SemaphoreType.DMA,
        ],
    )
    def k(feat_hbm, idx_hbm, table_hbm, out_hbm,
          idx_v, feat_v, rows_v, acc_v, gsem, fsem):
        wid = lax.axis_index("s") * _NC + lax.axis_index("c")
        base = wid * _B_PER_W
        fcopy = pltpu.async_copy(
            feat_hbm.at[pl.ds(base, _B_PER_W), :], feat_v, fsem)
        pltpu.sync_copy(idx_hbm.at[pl.ds(base, _B_PER_W)], idx_v)
        pltpu.async_copy(table_hbm.at[idx_v], rows_v, gsem).wait()
        fcopy.wait()

        def body(r, acc):
            for g in range(_GROUPS):
                f = feat_v[r, pl.ds(g * _NL, _NL)]
                c = rows_v[r, pl.ds(g * _NL, _NL)]
                d = f - c
                acc = acc + d * d
            return acc

        acc = lax.fori_loop(0, _B_PER_W, body, jnp.zeros((_NL,), jnp.float32))
        acc_v[...] = acc
        pltpu.sync_copy(acc_v, out_hbm.at[wid])

    return k(features, labels, centers)


@jax.jit
def kernel(features, labels, centers):
    idx = labels.astype(jnp.int32)
    parts = _partials(features, idx, centers)
    return (_LAMBDA_C / features.shape[0]) * jnp.sum(parts)
